# baseline (speedup 1.0000x reference)
import functools

import jax
import jax.numpy as jnp
from jax import lax
from jax.experimental import pallas as pl
from jax.experimental.pallas import tpu as pltpu
from jax.experimental.pallas import tpu_sc as plsc

BATCH = 16384
EMB = 32
HID = 128

_info = plsc.get_sparse_core_info()
_NC, _NS = _info.num_cores, _info.num_subcores
_NW = _NC * _NS
_BPW = BATCH // _NW
_CH = 128
_NCH = _BPW // _CH


def _gather_body(uidx_hbm, bidx_hbm, ut_hbm, bt_hbm, u_out, b_out,
                 uidx_v, bidx_v, urows_v, brows_v, sem):
    wid = lax.axis_index("s") * _NC + lax.axis_index("c")
    base = wid * _BPW
    pltpu.sync_copy(uidx_hbm.at[pl.ds(wid * _NCH, _NCH)], uidx_v)
    pltpu.sync_copy(bidx_hbm.at[pl.ds(wid * _NCH, _NCH)], bidx_v)
    copies = []
    for j in range(_NCH):
        copies.append(pltpu.async_copy(
            ut_hbm.at[uidx_v.at[j]], urows_v.at[pl.ds(j * _CH, _CH)], sem))
        copies.append(pltpu.async_copy(
            bt_hbm.at[bidx_v.at[j]], brows_v.at[pl.ds(j * _CH, _CH)], sem))
    for c in copies:
        c.wait()
    pltpu.sync_copy(urows_v, u_out.at[pl.ds(base, _BPW)])
    pltpu.sync_copy(brows_v, b_out.at[pl.ds(base, _BPW)])


_gather = pl.kernel(
    _gather_body,
    mesh=plsc.VectorSubcoreMesh(core_axis_name="c", subcore_axis_name="s"),
    out_type=[
        jax.ShapeDtypeStruct((BATCH, EMB), jnp.float32),
        jax.ShapeDtypeStruct((BATCH, EMB), jnp.float32),
    ],
    scratch_types=[
        pltpu.VMEM((_NCH, _CH), jnp.int32),
        pltpu.VMEM((_NCH, _CH), jnp.int32),
        pltpu.VMEM((_BPW, EMB), jnp.float32),
        pltpu.VMEM((_BPW, EMB), jnp.float32),
        pltpu.SemaphoreType.DMA,
    ],
    compiler_params=pltpu.CompilerParams(use_tc_tiling_on_sc=False),
)

_BLK = 2048


def _mlp_body(u_ref, bk_ref, w1u_ref, w1b_ref, b1_ref, w2_ref, b2_ref, o_ref):
    h = jnp.dot(u_ref[...], w1u_ref[...], preferred_element_type=jnp.float32)
    h = h + jnp.dot(bk_ref[...], w1b_ref[...], preferred_element_type=jnp.float32)
    h = jnp.maximum(h + b1_ref[...], 0.0)
    o_ref[...] = jnp.sum(h * w2_ref[...], axis=1) + b2_ref[0, 0]


def _mlp(u, bk, w1u, w1b, b1, w2, b2):
    grid = BATCH // _BLK
    return pl.pallas_call(
        _mlp_body,
        grid=(grid,),
        in_specs=[
            pl.BlockSpec((_BLK, EMB), lambda i: (i, 0)),
            pl.BlockSpec((_BLK, EMB), lambda i: (i, 0)),
            pl.BlockSpec((EMB, HID), lambda i: (0, 0)),
            pl.BlockSpec((EMB, HID), lambda i: (0, 0)),
            pl.BlockSpec((1, HID), lambda i: (0, 0)),
            pl.BlockSpec((1, HID), lambda i: (0, 0)),
            pl.BlockSpec(memory_space=pltpu.SMEM),
        ],
        out_specs=pl.BlockSpec((_BLK,), lambda i: (i,)),
        out_shape=jax.ShapeDtypeStruct((BATCH,), jnp.float32),
    )(u, bk, w1u, w1b, b1, w2, b2)


def kernel(user, book, user_table, book_table, W1, b1, W2, b2):
    uidx = user.astype(jnp.int32).reshape(_NW * _NCH, _CH)
    bidx = book.astype(jnp.int32).reshape(_NW * _NCH, _CH)
    u, bk = _gather(uidx, bidx, user_table, book_table)
    w1t = W1.T
    w1u = w1t[:EMB]
    w1b = w1t[EMB:]
    b1r = b1.reshape(1, HID)
    w2r = W2.reshape(1, HID)
    b2r = b2.reshape(1, 1)
    return _mlp(u, bk, w1u, w1b, b1r, w2r, b2r)
